# baseline (speedup 1.0000x reference)
import functools

import jax
import jax.numpy as jnp
from jax import lax
from jax.experimental import pallas as pl
from jax.experimental.pallas import tpu as pltpu
from jax.experimental.pallas import tpu_sc as plsc

_H = 768
_E = 64
_TOKENS = 32768
_EPS = 1e-6
_ROOT = _H ** -0.5

_NP = 8
_PS = _E // _NP

_NC, _NS, _L = 2, 16, 16
_NW = _NC * _NS
_TPW = _TOKENS // _NW
_NSLAB = 2
_GU = 4


def _logits_body(x_ref, w_ref, s_ref, v_ref, i_ref):
    x = x_ref[...]
    ms = jnp.mean(x * x, axis=1, keepdims=True)
    n = x * jax.lax.rsqrt(ms + _EPS)
    n = n * jnp.float32(_ROOT)
    n = n * s_ref[...]
    nb = n.astype(jnp.bfloat16)
    wb = w_ref[...].astype(jnp.bfloat16)
    lt = jax.lax.dot_general(
        wb, nb, (((1,), (1,)), ((), ())),
        preferred_element_type=jnp.float32)
    tb = lt.shape[1]
    lt3 = lt.reshape(_NP, _PS, tb)
    li = jax.lax.broadcasted_iota(jnp.int32, (_NP, _PS, tb), 1)
    m1p = jnp.max(lt3, axis=1, keepdims=True)
    l1p = jnp.min(jnp.where(lt3 == m1p, li, _PS), axis=1, keepdims=True)
    masked = jnp.where(li == l1p, -jnp.inf, lt3)
    m2p = jnp.max(masked, axis=1, keepdims=True)
    l2p = jnp.min(jnp.where(masked == m2p, li, _PS), axis=1, keepdims=True)
    pbase = jax.lax.broadcasted_iota(jnp.int32, (_NP, 1, tb), 0) * _PS
    g1p = (pbase + l1p).reshape(_NP, tb)
    g2p = (pbase + l2p).reshape(_NP, tb)
    v_ref[...] = jnp.concatenate(
        [m1p.reshape(_NP, tb), m2p.reshape(_NP, tb)], axis=0)
    i_ref[...] = jnp.concatenate([g1p, g2p], axis=0)


_sc_mesh = plsc.VectorSubcoreMesh(core_axis_name="c", subcore_axis_name="s")

_TPS = _TPW // _NSLAB


@functools.partial(
    pl.kernel,
    mesh=_sc_mesh,
    out_type=[jax.ShapeDtypeStruct((2, _TOKENS), jnp.float32),
              jax.ShapeDtypeStruct((2, _TOKENS), jnp.int32)],
    scratch_types=[pltpu.VMEM((2 * _NP, _TPW), jnp.float32),
                   pltpu.VMEM((2 * _NP, _TPW), jnp.int32),
                   pltpu.VMEM((2, _TPW), jnp.float32),
                   pltpu.VMEM((2, _TPW), jnp.int32),
                   pltpu.SemaphoreType.DMA((2 * _NSLAB,))],
)
def _sc_topk(v_hbm, x_hbm, w_hbm, i_hbm, v_v, x_v, w_v, i_v, sems):
    wid = lax.axis_index("s") * _NC + lax.axis_index("c")
    base = wid * _TPW

    copies = []
    for s in range(_NSLAB):
        copies.append((
            pltpu.async_copy(
                v_hbm.at[:, pl.ds(base + s * _TPS, _TPS)],
                v_v.at[:, pl.ds(s * _TPS, _TPS)],
                sems.at[2 * s]),
            pltpu.async_copy(
                x_hbm.at[:, pl.ds(base + s * _TPS, _TPS)],
                x_v.at[:, pl.ds(s * _TPS, _TPS)],
                sems.at[2 * s + 1]),
        ))

    nblk = _TPS // (_GU * _L)

    for s in range(_NSLAB):
        copies[s][0].wait()
        copies[s][1].wait()

        def per_block(b, _, s=s):
            col0 = s * _TPS + b * (_GU * _L)
            for u in range(_GU):
                col = col0 + u * _L
                m1 = v_v[0, pl.ds(col, _L)]
                i1 = x_v[0, pl.ds(col, _L)]
                sw = v_v[_NP, pl.ds(col, _L)]
                si = x_v[_NP, pl.ds(col, _L)]
                m2 = jnp.full((_L,), -jnp.inf, jnp.float32)
                i2 = jnp.zeros((_L,), jnp.int32)
                for p in range(1, _NP):
                    v = v_v[p, pl.ds(col, _L)]
                    vi = x_v[p, pl.ds(col, _L)]
                    sv = v_v[_NP + p, pl.ds(col, _L)]
                    svi = x_v[_NP + p, pl.ds(col, _L)]
                    gt1 = v > m1
                    gt2 = v > m2
                    m2n = jnp.where(gt1, m1, jnp.where(gt2, v, m2))
                    i2n = jnp.where(gt1, i1, jnp.where(gt2, vi, i2))
                    m1 = jnp.where(gt1, v, m1)
                    i1 = jnp.where(gt1, vi, i1)
                    sw = jnp.where(gt1, sv, sw)
                    si = jnp.where(gt1, svi, si)
                    m2, i2 = m2n, i2n
                take = (sw > m2) | ((sw == m2) & (si < i2))
                m2 = jnp.where(take, sw, m2)
                i2 = jnp.where(take, si, i2)
                ex = jnp.exp(m2 - m1)
                w2 = ex / (1.0 + ex)
                w1 = 1.0 - w2
                w_v[0, pl.ds(col, _L)] = w1
                w_v[1, pl.ds(col, _L)] = w2
                i_v[0, pl.ds(col, _L)] = i1
                i_v[1, pl.ds(col, _L)] = i2
            return 0

        lax.fori_loop(0, nblk, functools.partial(per_block, s=s), 0)

    pltpu.sync_copy(w_v, w_hbm.at[:, pl.ds(base, _TPW)])
    pltpu.sync_copy(i_v, i_hbm.at[:, pl.ds(base, _TPW)])


def kernel(hidden_states, W, scale):
    Tb = 8192
    vals, idxs = pl.pallas_call(
        _logits_body,
        grid=(_TOKENS // Tb,),
        in_specs=[
            pl.BlockSpec((Tb, _H), lambda i: (i, 0)),
            pl.BlockSpec((_E, _H), lambda i: (0, 0)),
            pl.BlockSpec((1, _H), lambda i: (0, 0)),
        ],
        out_specs=[
            pl.BlockSpec((2 * _NP, Tb), lambda i: (0, i)),
            pl.BlockSpec((2 * _NP, Tb), lambda i: (0, i)),
        ],
        out_shape=[
            jax.ShapeDtypeStruct((2 * _NP, _TOKENS), jnp.float32),
            jax.ShapeDtypeStruct((2 * _NP, _TOKENS), jnp.int32),
        ],
        compiler_params=pltpu.CompilerParams(
            dimension_semantics=("arbitrary",)),
    )(hidden_states, W, scale.reshape(1, _H))
    w2d, i2d = _sc_topk(vals, idxs)
    return (w2d.T, i2d.T)


# speedup vs baseline: 1.1024x; 1.1024x over previous
import functools

import jax
import jax.numpy as jnp
from jax import lax
from jax.experimental import pallas as pl
from jax.experimental.pallas import tpu as pltpu
from jax.experimental.pallas import tpu_sc as plsc

_H = 768
_E = 64
_TOKENS = 32768
_EPS = 1e-6
_ROOT = _H ** -0.5

_NP = 4
_PS = _E // _NP

_NC, _NS, _L = 2, 16, 16
_NW = _NC * _NS
_TPW = _TOKENS // _NW
_NSLAB = 2
_GU = 4


def _logits_body(x_ref, w_ref, s_ref, v_ref, i_ref):
    x = x_ref[...]
    ms = jnp.mean(x * x, axis=1, keepdims=True)
    n = x * jax.lax.rsqrt(ms + _EPS)
    n = n * jnp.float32(_ROOT)
    n = n * s_ref[...]
    nb = n.astype(jnp.bfloat16)
    wb = w_ref[...].astype(jnp.bfloat16)
    lt = jax.lax.dot_general(
        wb, nb, (((1,), (1,)), ((), ())),
        preferred_element_type=jnp.float32)
    tb = lt.shape[1]
    lt3 = lt.reshape(_NP, _PS, tb)
    li = jax.lax.broadcasted_iota(jnp.int32, (_NP, _PS, tb), 1)
    m1p = jnp.max(lt3, axis=1, keepdims=True)
    l1p = jnp.min(jnp.where(lt3 == m1p, li, _PS), axis=1, keepdims=True)
    masked = jnp.where(li == l1p, -jnp.inf, lt3)
    m2p = jnp.max(masked, axis=1, keepdims=True)
    l2p = jnp.min(jnp.where(masked == m2p, li, _PS), axis=1, keepdims=True)
    pbase = jax.lax.broadcasted_iota(jnp.int32, (_NP, 1, tb), 0) * _PS
    g1p = (pbase + l1p).reshape(_NP, tb)
    g2p = (pbase + l2p).reshape(_NP, tb)
    v_ref[...] = jnp.concatenate(
        [m1p.reshape(_NP, tb), m2p.reshape(_NP, tb)], axis=0)
    i_ref[...] = jnp.concatenate([g1p, g2p], axis=0)


_sc_mesh = plsc.VectorSubcoreMesh(core_axis_name="c", subcore_axis_name="s")

_TPS = _TPW // _NSLAB


@functools.partial(
    pl.kernel,
    mesh=_sc_mesh,
    out_type=[jax.ShapeDtypeStruct((2, _TOKENS), jnp.float32),
              jax.ShapeDtypeStruct((2, _TOKENS), jnp.int32)],
    scratch_types=[pltpu.VMEM((2 * _NP, _TPW), jnp.float32),
                   pltpu.VMEM((2 * _NP, _TPW), jnp.int32),
                   pltpu.VMEM((2, _TPW), jnp.float32),
                   pltpu.VMEM((2, _TPW), jnp.int32),
                   pltpu.SemaphoreType.DMA((2 * _NSLAB,))],
)
def _sc_topk(v_hbm, x_hbm, w_hbm, i_hbm, v_v, x_v, w_v, i_v, sems):
    wid = lax.axis_index("s") * _NC + lax.axis_index("c")
    base = wid * _TPW

    copies = []
    for s in range(_NSLAB):
        copies.append((
            pltpu.async_copy(
                v_hbm.at[:, pl.ds(base + s * _TPS, _TPS)],
                v_v.at[:, pl.ds(s * _TPS, _TPS)],
                sems.at[2 * s]),
            pltpu.async_copy(
                x_hbm.at[:, pl.ds(base + s * _TPS, _TPS)],
                x_v.at[:, pl.ds(s * _TPS, _TPS)],
                sems.at[2 * s + 1]),
        ))

    nblk = _TPS // (_GU * _L)

    for s in range(_NSLAB):
        copies[s][0].wait()
        copies[s][1].wait()

        def per_block(b, _, s=s):
            col0 = s * _TPS + b * (_GU * _L)
            for u in range(_GU):
                col = col0 + u * _L
                m1 = v_v[0, pl.ds(col, _L)]
                i1 = x_v[0, pl.ds(col, _L)]
                sw = v_v[_NP, pl.ds(col, _L)]
                si = x_v[_NP, pl.ds(col, _L)]
                m2 = jnp.full((_L,), -jnp.inf, jnp.float32)
                i2 = jnp.zeros((_L,), jnp.int32)
                for p in range(1, _NP):
                    v = v_v[p, pl.ds(col, _L)]
                    vi = x_v[p, pl.ds(col, _L)]
                    sv = v_v[_NP + p, pl.ds(col, _L)]
                    svi = x_v[_NP + p, pl.ds(col, _L)]
                    gt1 = v > m1
                    gt2 = v > m2
                    m2n = jnp.where(gt1, m1, jnp.where(gt2, v, m2))
                    i2n = jnp.where(gt1, i1, jnp.where(gt2, vi, i2))
                    m1 = jnp.where(gt1, v, m1)
                    i1 = jnp.where(gt1, vi, i1)
                    sw = jnp.where(gt1, sv, sw)
                    si = jnp.where(gt1, svi, si)
                    m2, i2 = m2n, i2n
                take = (sw > m2) | ((sw == m2) & (si < i2))
                m2 = jnp.where(take, sw, m2)
                i2 = jnp.where(take, si, i2)
                ex = jnp.exp(m2 - m1)
                w2 = ex / (1.0 + ex)
                w1 = 1.0 - w2
                w_v[0, pl.ds(col, _L)] = w1
                w_v[1, pl.ds(col, _L)] = w2
                i_v[0, pl.ds(col, _L)] = i1
                i_v[1, pl.ds(col, _L)] = i2
            return 0

        lax.fori_loop(0, nblk, functools.partial(per_block, s=s), 0)

    pltpu.sync_copy(w_v, w_hbm.at[:, pl.ds(base, _TPW)])
    pltpu.sync_copy(i_v, i_hbm.at[:, pl.ds(base, _TPW)])


def kernel(hidden_states, W, scale):
    Tb = 4096
    vals, idxs = pl.pallas_call(
        _logits_body,
        grid=(_TOKENS // Tb,),
        in_specs=[
            pl.BlockSpec((Tb, _H), lambda i: (i, 0)),
            pl.BlockSpec((_E, _H), lambda i: (0, 0)),
            pl.BlockSpec((1, _H), lambda i: (0, 0)),
        ],
        out_specs=[
            pl.BlockSpec((2 * _NP, Tb), lambda i: (0, i)),
            pl.BlockSpec((2 * _NP, Tb), lambda i: (0, i)),
        ],
        out_shape=[
            jax.ShapeDtypeStruct((2 * _NP, _TOKENS), jnp.float32),
            jax.ShapeDtypeStruct((2 * _NP, _TOKENS), jnp.int32),
        ],
        compiler_params=pltpu.CompilerParams(
            dimension_semantics=("arbitrary",)),
    )(hidden_states, W, scale.reshape(1, _H))
    w2d, i2d = _sc_topk(vals, idxs)
    return (w2d.T, i2d.T)


# speedup vs baseline: 1.1414x; 1.0354x over previous
import functools

import jax
import jax.numpy as jnp
from jax import lax
from jax.experimental import pallas as pl
from jax.experimental.pallas import tpu as pltpu
from jax.experimental.pallas import tpu_sc as plsc

_H = 768
_E = 64
_TOKENS = 32768
_EPS = 1e-6
_ROOT = _H ** -0.5

_NP = 2
_PS = _E // _NP

_NC, _NS, _L = 2, 16, 16
_NW = _NC * _NS
_TPW = _TOKENS // _NW
_NSLAB = 2
_GU = 4


def _logits_body(x_ref, w_ref, s_ref, v_ref, i_ref):
    x = x_ref[...]
    ms = jnp.mean(x * x, axis=1, keepdims=True)
    n = x * jax.lax.rsqrt(ms + _EPS)
    n = n * jnp.float32(_ROOT)
    n = n * s_ref[...]
    nb = n.astype(jnp.bfloat16)
    wb = w_ref[...].astype(jnp.bfloat16)
    lt = jax.lax.dot_general(
        wb, nb, (((1,), (1,)), ((), ())),
        preferred_element_type=jnp.float32)
    tb = lt.shape[1]
    lt3 = lt.reshape(_NP, _PS, tb)
    li = jax.lax.broadcasted_iota(jnp.int32, (_NP, _PS, tb), 1)
    m1p = jnp.max(lt3, axis=1, keepdims=True)
    l1p = jnp.min(jnp.where(lt3 == m1p, li, _PS), axis=1, keepdims=True)
    masked = jnp.where(li == l1p, -jnp.inf, lt3)
    m2p = jnp.max(masked, axis=1, keepdims=True)
    l2p = jnp.min(jnp.where(masked == m2p, li, _PS), axis=1, keepdims=True)
    pbase = jax.lax.broadcasted_iota(jnp.int32, (_NP, 1, tb), 0) * _PS
    g1p = (pbase + l1p).reshape(_NP, tb)
    g2p = (pbase + l2p).reshape(_NP, tb)
    v_ref[...] = jnp.concatenate(
        [m1p.reshape(_NP, tb), m2p.reshape(_NP, tb)], axis=0)
    i_ref[...] = jnp.concatenate([g1p, g2p], axis=0)


_sc_mesh = plsc.VectorSubcoreMesh(core_axis_name="c", subcore_axis_name="s")

_TPS = _TPW // _NSLAB


@functools.partial(
    pl.kernel,
    mesh=_sc_mesh,
    out_type=[jax.ShapeDtypeStruct((2, _TOKENS), jnp.float32),
              jax.ShapeDtypeStruct((2, _TOKENS), jnp.int32)],
    scratch_types=[pltpu.VMEM((2 * _NP, _TPW), jnp.float32),
                   pltpu.VMEM((2 * _NP, _TPW), jnp.int32),
                   pltpu.VMEM((2, _TPW), jnp.float32),
                   pltpu.VMEM((2, _TPW), jnp.int32),
                   pltpu.SemaphoreType.DMA((2 * _NSLAB,))],
)
def _sc_topk(v_hbm, x_hbm, w_hbm, i_hbm, v_v, x_v, w_v, i_v, sems):
    wid = lax.axis_index("s") * _NC + lax.axis_index("c")
    base = wid * _TPW

    copies = []
    for s in range(_NSLAB):
        copies.append((
            pltpu.async_copy(
                v_hbm.at[:, pl.ds(base + s * _TPS, _TPS)],
                v_v.at[:, pl.ds(s * _TPS, _TPS)],
                sems.at[2 * s]),
            pltpu.async_copy(
                x_hbm.at[:, pl.ds(base + s * _TPS, _TPS)],
                x_v.at[:, pl.ds(s * _TPS, _TPS)],
                sems.at[2 * s + 1]),
        ))

    nblk = _TPS // (_GU * _L)

    for s in range(_NSLAB):
        copies[s][0].wait()
        copies[s][1].wait()

        def per_block(b, _, s=s):
            col0 = s * _TPS + b * (_GU * _L)
            for u in range(_GU):
                col = col0 + u * _L
                m1 = v_v[0, pl.ds(col, _L)]
                i1 = x_v[0, pl.ds(col, _L)]
                sw = v_v[_NP, pl.ds(col, _L)]
                si = x_v[_NP, pl.ds(col, _L)]
                m2 = jnp.full((_L,), -jnp.inf, jnp.float32)
                i2 = jnp.zeros((_L,), jnp.int32)
                for p in range(1, _NP):
                    v = v_v[p, pl.ds(col, _L)]
                    vi = x_v[p, pl.ds(col, _L)]
                    sv = v_v[_NP + p, pl.ds(col, _L)]
                    svi = x_v[_NP + p, pl.ds(col, _L)]
                    gt1 = v > m1
                    gt2 = v > m2
                    m2n = jnp.where(gt1, m1, jnp.where(gt2, v, m2))
                    i2n = jnp.where(gt1, i1, jnp.where(gt2, vi, i2))
                    m1 = jnp.where(gt1, v, m1)
                    i1 = jnp.where(gt1, vi, i1)
                    sw = jnp.where(gt1, sv, sw)
                    si = jnp.where(gt1, svi, si)
                    m2, i2 = m2n, i2n
                take = (sw > m2) | ((sw == m2) & (si < i2))
                m2 = jnp.where(take, sw, m2)
                i2 = jnp.where(take, si, i2)
                ex = jnp.exp(m2 - m1)
                w2 = ex / (1.0 + ex)
                w1 = 1.0 - w2
                w_v[0, pl.ds(col, _L)] = w1
                w_v[1, pl.ds(col, _L)] = w2
                i_v[0, pl.ds(col, _L)] = i1
                i_v[1, pl.ds(col, _L)] = i2
            return 0

        lax.fori_loop(0, nblk, functools.partial(per_block, s=s), 0)

    pltpu.sync_copy(w_v, w_hbm.at[:, pl.ds(base, _TPW)])
    pltpu.sync_copy(i_v, i_hbm.at[:, pl.ds(base, _TPW)])


def kernel(hidden_states, W, scale):
    Tb = 4096
    vals, idxs = pl.pallas_call(
        _logits_body,
        grid=(_TOKENS // Tb,),
        in_specs=[
            pl.BlockSpec((Tb, _H), lambda i: (i, 0)),
            pl.BlockSpec((_E, _H), lambda i: (0, 0)),
            pl.BlockSpec((1, _H), lambda i: (0, 0)),
        ],
        out_specs=[
            pl.BlockSpec((2 * _NP, Tb), lambda i: (0, i)),
            pl.BlockSpec((2 * _NP, Tb), lambda i: (0, i)),
        ],
        out_shape=[
            jax.ShapeDtypeStruct((2 * _NP, _TOKENS), jnp.float32),
            jax.ShapeDtypeStruct((2 * _NP, _TOKENS), jnp.int32),
        ],
        compiler_params=pltpu.CompilerParams(
            dimension_semantics=("arbitrary",)),
    )(hidden_states, W, scale.reshape(1, _H))
    w2d, i2d = _sc_topk(vals, idxs)
    return (w2d.T, i2d.T)


# speedup vs baseline: 1.1484x; 1.0061x over previous
import functools

import jax
import jax.numpy as jnp
from jax import lax
from jax.experimental import pallas as pl
from jax.experimental.pallas import tpu as pltpu
from jax.experimental.pallas import tpu_sc as plsc

_H = 768
_E = 64
_TOKENS = 32768
_EPS = 1e-6
_ROOT = _H ** -0.5

_NP = 2
_PS = _E // _NP

_NC, _NS, _L = 2, 16, 16
_NW = _NC * _NS
_TPW = _TOKENS // _NW
_NSLAB = 1
_GU = 4


def _logits_body(x_ref, w_ref, s_ref, v_ref, i_ref):
    x = x_ref[...]
    ms = jnp.mean(x * x, axis=1, keepdims=True)
    n = x * jax.lax.rsqrt(ms + _EPS)
    n = n * jnp.float32(_ROOT)
    n = n * s_ref[...]
    nb = n.astype(jnp.bfloat16)
    wb = w_ref[...].astype(jnp.bfloat16)
    lt = jax.lax.dot_general(
        wb, nb, (((1,), (1,)), ((), ())),
        preferred_element_type=jnp.float32)
    tb = lt.shape[1]
    lt3 = lt.reshape(_NP, _PS, tb)
    li = jax.lax.broadcasted_iota(jnp.int32, (_NP, _PS, tb), 1)
    m1p = jnp.max(lt3, axis=1, keepdims=True)
    l1p = jnp.min(jnp.where(lt3 == m1p, li, _PS), axis=1, keepdims=True)
    masked = jnp.where(li == l1p, -jnp.inf, lt3)
    m2p = jnp.max(masked, axis=1, keepdims=True)
    l2p = jnp.min(jnp.where(masked == m2p, li, _PS), axis=1, keepdims=True)
    pbase = jax.lax.broadcasted_iota(jnp.int32, (_NP, 1, tb), 0) * _PS
    g1p = (pbase + l1p).reshape(_NP, tb)
    g2p = (pbase + l2p).reshape(_NP, tb)
    v_ref[...] = jnp.concatenate(
        [m1p.reshape(_NP, tb), m2p.reshape(_NP, tb)], axis=0)
    i_ref[...] = jnp.concatenate([g1p, g2p], axis=0)


_sc_mesh = plsc.VectorSubcoreMesh(core_axis_name="c", subcore_axis_name="s")

_TPS = _TPW // _NSLAB


@functools.partial(
    pl.kernel,
    mesh=_sc_mesh,
    out_type=[jax.ShapeDtypeStruct((2, _TOKENS), jnp.float32),
              jax.ShapeDtypeStruct((2, _TOKENS), jnp.int32)],
    scratch_types=[pltpu.VMEM((2 * _NP, _TPW), jnp.float32),
                   pltpu.VMEM((2 * _NP, _TPW), jnp.int32),
                   pltpu.VMEM((2, _TPW), jnp.float32),
                   pltpu.VMEM((2, _TPW), jnp.int32),
                   pltpu.SemaphoreType.DMA((2 * _NSLAB,))],
)
def _sc_topk(v_hbm, x_hbm, w_hbm, i_hbm, v_v, x_v, w_v, i_v, sems):
    wid = lax.axis_index("s") * _NC + lax.axis_index("c")
    base = wid * _TPW

    copies = []
    for s in range(_NSLAB):
        copies.append((
            pltpu.async_copy(
                v_hbm.at[:, pl.ds(base + s * _TPS, _TPS)],
                v_v.at[:, pl.ds(s * _TPS, _TPS)],
                sems.at[2 * s]),
            pltpu.async_copy(
                x_hbm.at[:, pl.ds(base + s * _TPS, _TPS)],
                x_v.at[:, pl.ds(s * _TPS, _TPS)],
                sems.at[2 * s + 1]),
        ))

    nblk = _TPS // (_GU * _L)

    for s in range(_NSLAB):
        copies[s][0].wait()
        copies[s][1].wait()

        def per_block(b, _, s=s):
            col0 = s * _TPS + b * (_GU * _L)
            for u in range(_GU):
                col = col0 + u * _L
                m1 = v_v[0, pl.ds(col, _L)]
                i1 = x_v[0, pl.ds(col, _L)]
                sw = v_v[_NP, pl.ds(col, _L)]
                si = x_v[_NP, pl.ds(col, _L)]
                m2 = jnp.full((_L,), -jnp.inf, jnp.float32)
                i2 = jnp.zeros((_L,), jnp.int32)
                for p in range(1, _NP):
                    v = v_v[p, pl.ds(col, _L)]
                    vi = x_v[p, pl.ds(col, _L)]
                    sv = v_v[_NP + p, pl.ds(col, _L)]
                    svi = x_v[_NP + p, pl.ds(col, _L)]
                    gt1 = v > m1
                    gt2 = v > m2
                    m2n = jnp.where(gt1, m1, jnp.where(gt2, v, m2))
                    i2n = jnp.where(gt1, i1, jnp.where(gt2, vi, i2))
                    m1 = jnp.where(gt1, v, m1)
                    i1 = jnp.where(gt1, vi, i1)
                    sw = jnp.where(gt1, sv, sw)
                    si = jnp.where(gt1, svi, si)
                    m2, i2 = m2n, i2n
                take = (sw > m2) | ((sw == m2) & (si < i2))
                m2 = jnp.where(take, sw, m2)
                i2 = jnp.where(take, si, i2)
                ex = jnp.exp(m2 - m1)
                w2 = ex / (1.0 + ex)
                w1 = 1.0 - w2
                w_v[0, pl.ds(col, _L)] = w1
                w_v[1, pl.ds(col, _L)] = w2
                i_v[0, pl.ds(col, _L)] = i1
                i_v[1, pl.ds(col, _L)] = i2
            return 0

        lax.fori_loop(0, nblk, functools.partial(per_block, s=s), 0)

    pltpu.sync_copy(w_v, w_hbm.at[:, pl.ds(base, _TPW)])
    pltpu.sync_copy(i_v, i_hbm.at[:, pl.ds(base, _TPW)])


def kernel(hidden_states, W, scale):
    Tb = 4096
    vals, idxs = pl.pallas_call(
        _logits_body,
        grid=(_TOKENS // Tb,),
        in_specs=[
            pl.BlockSpec((Tb, _H), lambda i: (i, 0)),
            pl.BlockSpec((_E, _H), lambda i: (0, 0)),
            pl.BlockSpec((1, _H), lambda i: (0, 0)),
        ],
        out_specs=[
            pl.BlockSpec((2 * _NP, Tb), lambda i: (0, i)),
            pl.BlockSpec((2 * _NP, Tb), lambda i: (0, i)),
        ],
        out_shape=[
            jax.ShapeDtypeStruct((2 * _NP, _TOKENS), jnp.float32),
            jax.ShapeDtypeStruct((2 * _NP, _TOKENS), jnp.int32),
        ],
        compiler_params=pltpu.CompilerParams(
            dimension_semantics=("arbitrary",)),
    )(hidden_states, W, scale.reshape(1, _H))
    w2d, i2d = _sc_topk(vals, idxs)
    return (w2d.T, i2d.T)
